# baseline (speedup 1.0000x reference)
import functools

import jax
import jax.numpy as jnp
from jax import lax
from jax.experimental import pallas as pl
from jax.experimental.pallas import tpu as pltpu
from jax.experimental.pallas import tpu_sc as plsc

KNN = 20
ROWS = 128


CHUNK = 128
PRE = 6


def _pairwise(q, kt):
    inner2 = -2.0 * lax.dot_general(
        q, kt, (((1,), (0,)), ((), ())),
        preferred_element_type=jnp.float32)
    xx_q = jnp.sum(q * q, axis=1, keepdims=True)
    xx_k = jnp.sum(kt * kt, axis=0, keepdims=True)
    return (-xx_q - inner2) - xx_k


def _topk_body(p_ref, pt_ref, idx_ref):
    q = p_ref[0]
    kt = pt_ref[0]
    n = kt.shape[1]
    rows = q.shape[0]
    nch = n // CHUNK

    pw = _pairwise(q, kt)

    w_iota = lax.broadcasted_iota(jnp.int32, (rows, CHUNK), 1)
    vs, is_, tlast = [], [], []
    for c in range(nch):
        ch = pw[:, c * CHUNK:(c + 1) * CHUNK]
        vc, ic = [], []
        for _ in range(PRE):
            m = jnp.max(ch, axis=1, keepdims=True)
            candw = jnp.where(ch == m, w_iota, CHUNK)
            sw = jnp.min(candw, axis=1, keepdims=True)
            vc.append(m)
            ic.append(sw + c * CHUNK)
            ch = jnp.where(w_iota == sw, -jnp.inf, ch)
        vs.append(jnp.concatenate(vc, axis=1))
        is_.append(jnp.concatenate(ic, axis=1))
        tlast.append(vc[-1])

    vcand = jnp.concatenate(vs, axis=1)
    icand = jnp.concatenate(is_, axis=1)
    sels = []
    v20 = None
    for i in range(KNN):
        m = jnp.max(vcand, axis=1, keepdims=True)
        cand = jnp.where(vcand == m, icand, n)
        si = jnp.min(cand, axis=1, keepdims=True)
        sels.append(si)
        if i == KNN - 1:
            v20 = m
        vcand = jnp.where(icand == si, -jnp.inf, vcand)
    idx_ref[0] = jnp.concatenate(sels, axis=1)

    invalid = jnp.any(jnp.concatenate(tlast, axis=1) >= v20)

    @pl.when(invalid)
    def _fallback():
        pwf = _pairwise(q, kt)
        col = lax.broadcasted_iota(jnp.int32, pwf.shape, 1)
        fsels = []
        for _ in range(KNN):
            m = jnp.max(pwf, axis=1, keepdims=True)
            cand = jnp.where(pwf == m, col, n)
            sel = jnp.min(cand, axis=1, keepdims=True)
            fsels.append(sel)
            pwf = jnp.where(col == sel, -jnp.inf, pwf)
        idx_ref[0] = jnp.concatenate(fsels, axis=1)


def _topk_call(p, pt):
    b, n, _ = p.shape
    return pl.pallas_call(
        _topk_body,
        grid=(b, n // ROWS),
        in_specs=[
            pl.BlockSpec((1, ROWS, 3), lambda i, r: (i, r, 0)),
            pl.BlockSpec((1, 3, n), lambda i, r: (i, 0, 0)),
        ],
        out_specs=pl.BlockSpec((1, ROWS, KNN), lambda i, r: (i, r, 0)),
        out_shape=jax.ShapeDtypeStruct((b, n, KNN), jnp.int32),
    )(p, pt)


def _features_call(pt_flat, idx_flat, b, n):
    nk = n * KNN
    info = plsc.get_sparse_core_info()
    nw = info.num_cores * info.num_subcores
    tiles_per_b = nw // b
    per_tile = nk // tiles_per_b
    chunk = 2048
    mesh = plsc.VectorSubcoreMesh(core_axis_name="c", subcore_axis_name="s")

    @functools.partial(
        pl.kernel,
        out_type=jax.ShapeDtypeStruct((b * 9 * nk,), jnp.float32),
        mesh=mesh,
        compiler_params=pltpu.CompilerParams(needs_layout_passes=False),
        scratch_types=[
            pltpu.VMEM((n,), jnp.float32),
            pltpu.VMEM((n,), jnp.float32),
            pltpu.VMEM((n,), jnp.float32),
            pltpu.VMEM((per_tile,), jnp.int32),
        ] + [pltpu.VMEM((chunk,), jnp.float32) for _ in range(9)],
    )
    def sc_kernel(pt_hbm, idx_hbm, out_hbm, px, py, pz, idxv, *ob):
        wid = lax.axis_index("s") * info.num_cores + lax.axis_index("c")
        bi = wid // tiles_per_b
        base = (wid % tiles_per_b) * per_tile

        pltpu.sync_copy(pt_hbm.at[pl.ds((bi * 3 + 0) * n, n)], px)
        pltpu.sync_copy(pt_hbm.at[pl.ds((bi * 3 + 1) * n, n)], py)
        pltpu.sync_copy(pt_hbm.at[pl.ds((bi * 3 + 2) * n, n)], pz)
        pltpu.sync_copy(idx_hbm.at[pl.ds(bi * nk + base, per_tile)], idxv)

        lane = jnp.arange(16, dtype=jnp.int32)

        for c in range(per_tile // chunk):
            def step(s, _, c=c):
                off = c * chunk + s * 16
                nidx = idxv[pl.ds(off, 16)]
                qidx = (base + off + lane) // KNN
                qx = plsc.load_gather(px, [qidx])
                qy = plsc.load_gather(py, [qidx])
                qz = plsc.load_gather(pz, [qidx])
                nx = plsc.load_gather(px, [nidx])
                ny = plsc.load_gather(py, [nidx])
                nz = plsc.load_gather(pz, [nidx])
                loc = pl.ds(s * 16, 16)
                ob[0][loc] = nx - qx
                ob[1][loc] = ny - qy
                ob[2][loc] = nz - qz
                ob[3][loc] = qx
                ob[4][loc] = qy
                ob[5][loc] = qz
                ob[6][loc] = qy * nz - qz * ny
                ob[7][loc] = qz * nx - qx * nz
                ob[8][loc] = qx * ny - qy * nx
                return 0

            lax.fori_loop(0, chunk // 16, step, 0)
            for j in range(9):
                pltpu.sync_copy(
                    ob[j],
                    out_hbm.at[pl.ds((bi * 9 + j) * nk + base + c * chunk,
                                     chunk)])

    return sc_kernel(pt_flat, idx_flat)


def kernel(p):
    b, n, _ = p.shape
    pt = jnp.transpose(p, (0, 2, 1))
    idx = _topk_call(p, pt)
    out = _features_call(pt.reshape(-1), idx.reshape(-1), b, n)
    return out.reshape(b, 3, 3, n, KNN)


# speedup vs baseline: 4.2542x; 4.2542x over previous
import functools

import jax
import jax.numpy as jnp
from jax import lax
from jax.experimental import pallas as pl
from jax.experimental.pallas import tpu as pltpu
from jax.experimental.pallas import tpu_sc as plsc

KNN = 20
ROWS = 512


CHUNK = 128
PRE = 6


def _pairwise(q, kt):
    inner2 = -2.0 * lax.dot_general(
        q, kt, (((1,), (0,)), ((), ())),
        preferred_element_type=jnp.float32)
    xx_q = jnp.sum(q * q, axis=1, keepdims=True)
    xx_k = jnp.sum(kt * kt, axis=0, keepdims=True)
    return (-xx_q - inner2) - xx_k


def _topk_body(p_ref, pt_ref, idx_ref):
    q = p_ref[0]
    kt = pt_ref[0]
    n = kt.shape[1]

    pw = _pairwise(q, kt)
    col = lax.broadcasted_iota(jnp.int32, pw.shape, 1)
    sels = []
    for _ in range(KNN):
        m = jnp.max(pw, axis=1, keepdims=True)
        cand = jnp.where(pw == m, col, n)
        sel = jnp.min(cand, axis=1, keepdims=True)
        sels.append(sel)
        pw = jnp.where(cand == sel, -jnp.inf, pw)
    idx_ref[0] = jnp.concatenate(sels, axis=1)


def _topk_call(p, pt):
    b, n, _ = p.shape
    return pl.pallas_call(
        _topk_body,
        grid=(b, n // ROWS),
        in_specs=[
            pl.BlockSpec((1, ROWS, 3), lambda i, r: (i, r, 0)),
            pl.BlockSpec((1, 3, n), lambda i, r: (i, 0, 0)),
        ],
        out_specs=pl.BlockSpec((1, ROWS, KNN), lambda i, r: (i, r, 0)),
        out_shape=jax.ShapeDtypeStruct((b, n, KNN), jnp.int32),
    )(p, pt)


def _features_call(pt_flat, idx_flat, b, n):
    nk = n * KNN
    info = plsc.get_sparse_core_info()
    nw = info.num_cores * info.num_subcores
    tiles_per_b = nw // b
    per_tile = nk // tiles_per_b
    chunk = 2048
    mesh = plsc.VectorSubcoreMesh(core_axis_name="c", subcore_axis_name="s")

    @functools.partial(
        pl.kernel,
        out_type=jax.ShapeDtypeStruct((b * 9 * nk,), jnp.float32),
        mesh=mesh,
        compiler_params=pltpu.CompilerParams(needs_layout_passes=False),
        scratch_types=[
            pltpu.VMEM((n,), jnp.float32),
            pltpu.VMEM((n,), jnp.float32),
            pltpu.VMEM((n,), jnp.float32),
            pltpu.VMEM((per_tile,), jnp.int32),
        ] + [pltpu.VMEM((chunk,), jnp.float32) for _ in range(9)],
    )
    def sc_kernel(pt_hbm, idx_hbm, out_hbm, px, py, pz, idxv, *ob):
        wid = lax.axis_index("s") * info.num_cores + lax.axis_index("c")
        bi = wid // tiles_per_b
        base = (wid % tiles_per_b) * per_tile

        pltpu.sync_copy(pt_hbm.at[pl.ds((bi * 3 + 0) * n, n)], px)
        pltpu.sync_copy(pt_hbm.at[pl.ds((bi * 3 + 1) * n, n)], py)
        pltpu.sync_copy(pt_hbm.at[pl.ds((bi * 3 + 2) * n, n)], pz)
        pltpu.sync_copy(idx_hbm.at[pl.ds(bi * nk + base, per_tile)], idxv)

        lane = jnp.arange(16, dtype=jnp.int32)

        for c in range(per_tile // chunk):
            def step(s, _, c=c):
                off = c * chunk + s * 16
                nidx = idxv[pl.ds(off, 16)]
                qidx = (base + off + lane) // KNN
                qx = plsc.load_gather(px, [qidx])
                qy = plsc.load_gather(py, [qidx])
                qz = plsc.load_gather(pz, [qidx])
                nx = plsc.load_gather(px, [nidx])
                ny = plsc.load_gather(py, [nidx])
                nz = plsc.load_gather(pz, [nidx])
                loc = pl.ds(s * 16, 16)
                ob[0][loc] = nx - qx
                ob[1][loc] = ny - qy
                ob[2][loc] = nz - qz
                ob[3][loc] = qx
                ob[4][loc] = qy
                ob[5][loc] = qz
                ob[6][loc] = qy * nz - qz * ny
                ob[7][loc] = qz * nx - qx * nz
                ob[8][loc] = qx * ny - qy * nx
                return 0

            lax.fori_loop(0, chunk // 16, step, 0)
            for j in range(9):
                pltpu.sync_copy(
                    ob[j],
                    out_hbm.at[pl.ds((bi * 9 + j) * nk + base + c * chunk,
                                     chunk)])

    return sc_kernel(pt_flat, idx_flat)


def kernel(p):
    b, n, _ = p.shape
    pt = jnp.transpose(p, (0, 2, 1))
    idx = _topk_call(p, pt)
    out = _features_call(pt.reshape(-1), idx.reshape(-1), b, n)
    return out.reshape(b, 3, 3, n, KNN)


# speedup vs baseline: 4.6133x; 1.0844x over previous
import functools

import jax
import jax.numpy as jnp
from jax import lax
from jax.experimental import pallas as pl
from jax.experimental.pallas import tpu as pltpu
from jax.experimental.pallas import tpu_sc as plsc

KNN = 20
ROWS = 512


CHUNK = 128
PRE = 6


def _pairwise(q, kt):
    inner2 = -2.0 * lax.dot_general(
        q, kt, (((1,), (0,)), ((), ())),
        preferred_element_type=jnp.float32)
    xx_q = jnp.sum(q * q, axis=1, keepdims=True)
    xx_k = jnp.sum(kt * kt, axis=0, keepdims=True)
    return (-xx_q - inner2) - xx_k


def _topk_body(p_ref, pt_ref, idx_ref):
    q = p_ref[0]
    kt = pt_ref[0]
    n = kt.shape[1]

    pw = _pairwise(q, kt)
    col = lax.broadcasted_iota(jnp.int32, pw.shape, 1)
    rows = q.shape[0]
    sels = []
    for _ in range(KNN):
        sel = jnp.argmax(pw, axis=1).astype(jnp.int32).reshape(rows, 1)
        sels.append(sel)
        pw = jnp.where(col == sel, -jnp.inf, pw)
    idx_ref[0] = jnp.concatenate(sels, axis=1)


def _topk_call(p, pt):
    b, n, _ = p.shape
    return pl.pallas_call(
        _topk_body,
        grid=(b, n // ROWS),
        in_specs=[
            pl.BlockSpec((1, ROWS, 3), lambda i, r: (i, r, 0)),
            pl.BlockSpec((1, 3, n), lambda i, r: (i, 0, 0)),
        ],
        out_specs=pl.BlockSpec((1, ROWS, KNN), lambda i, r: (i, r, 0)),
        out_shape=jax.ShapeDtypeStruct((b, n, KNN), jnp.int32),
    )(p, pt)


def _features_call(pt_flat, idx_flat, b, n):
    nk = n * KNN
    info = plsc.get_sparse_core_info()
    nw = info.num_cores * info.num_subcores
    tiles_per_b = nw // b
    per_tile = nk // tiles_per_b
    chunk = 2048
    mesh = plsc.VectorSubcoreMesh(core_axis_name="c", subcore_axis_name="s")

    @functools.partial(
        pl.kernel,
        out_type=jax.ShapeDtypeStruct((b * 9 * nk,), jnp.float32),
        mesh=mesh,
        compiler_params=pltpu.CompilerParams(needs_layout_passes=False),
        scratch_types=[
            pltpu.VMEM((n,), jnp.float32),
            pltpu.VMEM((n,), jnp.float32),
            pltpu.VMEM((n,), jnp.float32),
            pltpu.VMEM((per_tile,), jnp.int32),
        ] + [pltpu.VMEM((chunk,), jnp.float32) for _ in range(9)],
    )
    def sc_kernel(pt_hbm, idx_hbm, out_hbm, px, py, pz, idxv, *ob):
        wid = lax.axis_index("s") * info.num_cores + lax.axis_index("c")
        bi = wid // tiles_per_b
        base = (wid % tiles_per_b) * per_tile

        pltpu.sync_copy(pt_hbm.at[pl.ds((bi * 3 + 0) * n, n)], px)
        pltpu.sync_copy(pt_hbm.at[pl.ds((bi * 3 + 1) * n, n)], py)
        pltpu.sync_copy(pt_hbm.at[pl.ds((bi * 3 + 2) * n, n)], pz)
        pltpu.sync_copy(idx_hbm.at[pl.ds(bi * nk + base, per_tile)], idxv)

        lane = jnp.arange(16, dtype=jnp.int32)

        for c in range(per_tile // chunk):
            def step(s, _, c=c):
                off = c * chunk + s * 16
                nidx = idxv[pl.ds(off, 16)]
                qidx = (base + off + lane) // KNN
                qx = plsc.load_gather(px, [qidx])
                qy = plsc.load_gather(py, [qidx])
                qz = plsc.load_gather(pz, [qidx])
                nx = plsc.load_gather(px, [nidx])
                ny = plsc.load_gather(py, [nidx])
                nz = plsc.load_gather(pz, [nidx])
                loc = pl.ds(s * 16, 16)
                ob[0][loc] = nx - qx
                ob[1][loc] = ny - qy
                ob[2][loc] = nz - qz
                ob[3][loc] = qx
                ob[4][loc] = qy
                ob[5][loc] = qz
                ob[6][loc] = qy * nz - qz * ny
                ob[7][loc] = qz * nx - qx * nz
                ob[8][loc] = qx * ny - qy * nx
                return 0

            lax.fori_loop(0, chunk // 16, step, 0)
            for j in range(9):
                pltpu.sync_copy(
                    ob[j],
                    out_hbm.at[pl.ds((bi * 9 + j) * nk + base + c * chunk,
                                     chunk)])

    return sc_kernel(pt_flat, idx_flat)


def kernel(p):
    b, n, _ = p.shape
    pt = jnp.transpose(p, (0, 2, 1))
    idx = _topk_call(p, pt)
    out = _features_call(pt.reshape(-1), idx.reshape(-1), b, n)
    return out.reshape(b, 3, 3, n, KNN)


# speedup vs baseline: 4.8848x; 1.0588x over previous
import functools

import jax
import jax.numpy as jnp
from jax import lax
from jax.experimental import pallas as pl
from jax.experimental.pallas import tpu as pltpu
from jax.experimental.pallas import tpu_sc as plsc

KNN = 20
ROWS = 512


CHUNK = 512
PRE = 10


def _pairwise(q, kt):
    inner2 = -2.0 * lax.dot_general(
        q, kt, (((1,), (0,)), ((), ())),
        preferred_element_type=jnp.float32)
    xx_q = jnp.sum(q * q, axis=1, keepdims=True)
    xx_k = jnp.sum(kt * kt, axis=0, keepdims=True)
    return (-xx_q - inner2) - xx_k


def _topk_body(p_ref, pt_ref, idx_ref, vcand_ref, icand_ref):
    q = p_ref[0]
    kt = pt_ref[0]
    n = kt.shape[1]
    rows = q.shape[0]
    nch = n // CHUNK

    pw = _pairwise(q, kt)

    w_iota = lax.broadcasted_iota(jnp.int32, (rows, CHUNK), 1)
    tlast = []
    for c in range(nch):
        ch = pw[:, c * CHUNK:(c + 1) * CHUNK]
        m = None
        for r in range(PRE):
            m = jnp.max(ch, axis=1, keepdims=True)
            candw = jnp.where(ch == m, w_iota, CHUNK)
            sw = jnp.min(candw, axis=1, keepdims=True)
            vcand_ref[:, c * PRE + r] = m[:, 0]
            icand_ref[:, c * PRE + r] = sw[:, 0] + c * CHUNK
            if r != PRE - 1:
                ch = jnp.where(candw == sw, -jnp.inf, ch)
        tlast.append(m)

    vcand = vcand_ref[...]
    icand = icand_ref[...]
    sels = []
    v20 = None
    for i in range(KNN):
        m = jnp.max(vcand, axis=1, keepdims=True)
        cand = jnp.where(vcand == m, icand, n)
        si = jnp.min(cand, axis=1, keepdims=True)
        sels.append(si)
        if i == KNN - 1:
            v20 = m
        vcand = jnp.where(icand == si, -jnp.inf, vcand)
    idx_ref[0] = jnp.concatenate(sels, axis=1)

    invalid = jnp.any(jnp.concatenate(tlast, axis=1) >= v20)

    @pl.when(invalid)
    def _fallback():
        pwf = _pairwise(q, kt)
        col = lax.broadcasted_iota(jnp.int32, pwf.shape, 1)
        fsels = []
        for _ in range(KNN):
            sel = jnp.argmax(pwf, axis=1).astype(jnp.int32).reshape(rows, 1)
            fsels.append(sel)
            pwf = jnp.where(col == sel, -jnp.inf, pwf)
        idx_ref[0] = jnp.concatenate(fsels, axis=1)


def _topk_call(p, pt):
    b, n, _ = p.shape
    ncand = (n // CHUNK) * PRE
    return pl.pallas_call(
        _topk_body,
        grid=(b, n // ROWS),
        in_specs=[
            pl.BlockSpec((1, ROWS, 3), lambda i, r: (i, r, 0)),
            pl.BlockSpec((1, 3, n), lambda i, r: (i, 0, 0)),
        ],
        out_specs=pl.BlockSpec((1, ROWS, KNN), lambda i, r: (i, r, 0)),
        out_shape=jax.ShapeDtypeStruct((b, n, KNN), jnp.int32),
        scratch_shapes=[
            pltpu.VMEM((ROWS, ncand), jnp.float32),
            pltpu.VMEM((ROWS, ncand), jnp.int32),
        ],
        compiler_params=pltpu.CompilerParams(
            vmem_limit_bytes=100 * 1024 * 1024),
    )(p, pt)


def _features_call(pt_flat, idx_flat, b, n):
    nk = n * KNN
    info = plsc.get_sparse_core_info()
    nw = info.num_cores * info.num_subcores
    tiles_per_b = nw // b
    per_tile = nk // tiles_per_b
    chunk = 2048
    mesh = plsc.VectorSubcoreMesh(core_axis_name="c", subcore_axis_name="s")

    @functools.partial(
        pl.kernel,
        out_type=jax.ShapeDtypeStruct((b * 9 * nk,), jnp.float32),
        mesh=mesh,
        compiler_params=pltpu.CompilerParams(needs_layout_passes=False),
        scratch_types=[
            pltpu.VMEM((n,), jnp.float32),
            pltpu.VMEM((n,), jnp.float32),
            pltpu.VMEM((n,), jnp.float32),
            pltpu.VMEM((per_tile,), jnp.int32),
        ] + [pltpu.VMEM((chunk,), jnp.float32) for _ in range(9)],
    )
    def sc_kernel(pt_hbm, idx_hbm, out_hbm, px, py, pz, idxv, *ob):
        wid = lax.axis_index("s") * info.num_cores + lax.axis_index("c")
        bi = wid // tiles_per_b
        base = (wid % tiles_per_b) * per_tile

        pltpu.sync_copy(pt_hbm.at[pl.ds((bi * 3 + 0) * n, n)], px)
        pltpu.sync_copy(pt_hbm.at[pl.ds((bi * 3 + 1) * n, n)], py)
        pltpu.sync_copy(pt_hbm.at[pl.ds((bi * 3 + 2) * n, n)], pz)
        pltpu.sync_copy(idx_hbm.at[pl.ds(bi * nk + base, per_tile)], idxv)

        lane = jnp.arange(16, dtype=jnp.int32)

        for c in range(per_tile // chunk):
            def step(s, _, c=c):
                off = c * chunk + s * 16
                nidx = idxv[pl.ds(off, 16)]
                qidx = (base + off + lane) // KNN
                qx = plsc.load_gather(px, [qidx])
                qy = plsc.load_gather(py, [qidx])
                qz = plsc.load_gather(pz, [qidx])
                nx = plsc.load_gather(px, [nidx])
                ny = plsc.load_gather(py, [nidx])
                nz = plsc.load_gather(pz, [nidx])
                loc = pl.ds(s * 16, 16)
                ob[0][loc] = nx - qx
                ob[1][loc] = ny - qy
                ob[2][loc] = nz - qz
                ob[3][loc] = qx
                ob[4][loc] = qy
                ob[5][loc] = qz
                ob[6][loc] = qy * nz - qz * ny
                ob[7][loc] = qz * nx - qx * nz
                ob[8][loc] = qx * ny - qy * nx
                return 0

            lax.fori_loop(0, chunk // 16, step, 0)
            for j in range(9):
                pltpu.sync_copy(
                    ob[j],
                    out_hbm.at[pl.ds((bi * 9 + j) * nk + base + c * chunk,
                                     chunk)])

    return sc_kernel(pt_flat, idx_flat)


def kernel(p):
    b, n, _ = p.shape
    pt = jnp.transpose(p, (0, 2, 1))
    idx = _topk_call(p, pt)
    out = _features_call(pt.reshape(-1), idx.reshape(-1), b, n)
    return out.reshape(b, 3, 3, n, KNN)
